# Initial kernel scaffold; baseline (speedup 1.0000x reference)
import jax
import jax.numpy as jnp
from jax.experimental import pallas as pl


def kernel(node_features, hedge_features, node_idx, hedge_idx, W_node_self, W_hedge2node, b_node, W_hedge_self, W_node2hedge, b_hedge):
    raise NotImplementedError("write your pallas kernel here")


# speedup vs baseline: 6.0720x; 6.0720x over previous
import functools

import jax
import jax.numpy as jnp
from jax import lax
from jax.experimental import pallas as pl
from jax.experimental.pallas import tpu as pltpu
from jax.experimental.pallas import tpu_sc as plsc

N_CORES = 2
N_SUB = 16
NW = N_CORES * N_SUB
CH = 128
D = 128


def _sc_acc_body(nch_a, nch_b, acc_rows, table_hbm, src_hbm, dst_hbm,
                 zrow_hbm, acc_out, idx_d, is0, is1, gb0, gb1, acc_sh,
                 sg0, sg1, si0, si1):
    c = lax.axis_index("c")
    s = lax.axis_index("s")
    w = c * N_SUB + s
    stripe = acc_rows // N_SUB
    nch = jnp.where(c == 0, nch_a, nch_b)
    pltpu.sync_copy(zrow_hbm, acc_sh.at[pl.ds(s * stripe, stripe)])
    pltpu.sync_copy(dst_hbm.at[w], idx_d)
    plsc.subcore_barrier()

    gbufs = (gb0, gb1)
    isbufs = (is0, is1)
    sgs = (sg0, sg1)
    sis = (si0, si1)
    pltpu.sync_copy(src_hbm.at[w, 0], is0)
    pltpu.async_copy(table_hbm.at[is0], gb0, sg0)
    pltpu.async_copy(src_hbm.at[w, 1], is1, si1)

    def outer(t, carry):
        for b in range(2):
            j = t * 2 + b
            nb = 1 - b

            @pl.when(j + 1 < nch)
            def _start_next():
                pltpu.make_async_copy(src_hbm.at[w, j + 1],
                                      isbufs[nb], sis[nb]).wait()
                pltpu.async_copy(table_hbm.at[isbufs[nb]], gbufs[nb], sgs[nb])

            pltpu.make_async_copy(table_hbm.at[isbufs[b]],
                                  gbufs[b], sgs[b]).wait()

            @pl.when(j + 2 < nch)
            def _fetch_idx():
                pltpu.async_copy(src_hbm.at[w, j + 2], isbufs[b], sis[b])

            pltpu.sync_copy(gbufs[b], acc_sh.at[idx_d.at[j]], add=True)
        return carry

    lax.fori_loop(0, nch // 2, outer, 0)
    plsc.subcore_barrier()
    pltpu.sync_copy(acc_sh.at[pl.ds(s * stripe, stripe)],
                    acc_out.at[c, pl.ds(s * stripe, stripe)])


def _make_sc_acc(n_table, nch_a, nch_b, acc_rows):
    mesh = plsc.VectorSubcoreMesh(core_axis_name="c", subcore_axis_name="s")
    stripe = acc_rows // N_SUB
    return pl.kernel(
        functools.partial(_sc_acc_body, nch_a, nch_b, acc_rows),
        out_type=jax.ShapeDtypeStruct((N_CORES, acc_rows, D), jnp.float32),
        mesh=mesh,
        scratch_types=[
            pltpu.VMEM((nch_a, CH), jnp.int32),
            pltpu.VMEM((CH,), jnp.int32),
            pltpu.VMEM((CH,), jnp.int32),
            pltpu.VMEM((CH, D), jnp.float32),
            pltpu.VMEM((CH, D), jnp.float32),
            pltpu.VMEM_SHARED((acc_rows, D), jnp.float32),
            pltpu.SemaphoreType.DMA,
            pltpu.SemaphoreType.DMA,
            pltpu.SemaphoreType.DMA,
            pltpu.SemaphoreType.DMA,
        ],
        name=f"sc_acc_{n_table}",
    )


def _sc_deg_body(nch, acc_rows, dstn_hbm, dsth_hbm, degn_out, degh_out,
                 idx_n, idx_h, histn, histh):
    c = lax.axis_index("c")
    s = lax.axis_index("s")
    w = c * N_SUB + s
    zero = jnp.zeros((16,), jnp.float32)

    def initz(j, carry):
        histn[pl.ds(j * 16, 16)] = zero
        histh[pl.ds(j * 16, 16)] = zero
        return carry

    lax.fori_loop(0, acc_rows // 16, initz, 0)
    pltpu.sync_copy(dstn_hbm.at[w], idx_n)
    pltpu.sync_copy(dsth_hbm.at[w], idx_h)
    lanes = lax.iota(jnp.int32, 16)
    one = jnp.float32(1)
    zf = jnp.float32(0)

    def step(i, carry):
        j = i // (CH // 16)
        k = i % (CH // 16)
        vn = idx_n[j, pl.ds(k * 16, 16)]
        vh = idx_h[j, pl.ds(k * 16, 16)]
        for lane in range(16):
            v = vn[lane]
            plsc.addupdate(histn.at[pl.ds((v >> 4) << 4, 16)],
                           jnp.where(lanes == (v & 15), one, zf))
            u = vh[lane]
            plsc.addupdate(histh.at[pl.ds((u >> 4) << 4, 16)],
                           jnp.where(lanes == (u & 15), one, zf))
        return carry

    lax.fori_loop(0, nch * (CH // 16), step, 0)
    pltpu.sync_copy(histn, degn_out.at[w])
    pltpu.sync_copy(histh, degh_out.at[w])


def _make_sc_deg(nch, acc_rows):
    mesh = plsc.VectorSubcoreMesh(core_axis_name="c", subcore_axis_name="s")
    return pl.kernel(
        functools.partial(_sc_deg_body, nch, acc_rows),
        out_type=(
            jax.ShapeDtypeStruct((NW, acc_rows), jnp.float32),
            jax.ShapeDtypeStruct((NW, acc_rows), jnp.float32),
        ),
        mesh=mesh,
        scratch_types=[
            pltpu.VMEM((nch, CH), jnp.int32),
            pltpu.VMEM((nch, CH), jnp.int32),
            pltpu.VMEM((acc_rows,), jnp.float32),
            pltpu.VMEM((acc_rows,), jnp.float32),
        ],
        name="sc_deg",
    )


def _pre_body(nf, hf, w0, w1, bn, sn_out, th_out):
    sn_out[...] = jnp.dot(nf[...], w0[...],
                          preferred_element_type=jnp.float32) + bn[...]
    th_out[...] = jnp.dot(hf[...], w1[...], preferred_element_type=jnp.float32)


def _mid_body(sn, acc, deg, hf, wh, wn2h, bh, newn_out, tn_out, sh_out):
    a = acc[0] + acc[1]
    d = jnp.sum(deg[...], axis=1)[:, None]
    newn = jnp.tanh(sn[...] + a / jnp.maximum(d, 1.0))
    newn_out[...] = newn
    tn_out[...] = jnp.dot(newn, wn2h[...], preferred_element_type=jnp.float32)
    sh_out[...] = jnp.dot(hf[...], wh[...],
                          preferred_element_type=jnp.float32) + bh[...]


def _post_body(sh, acc, deg, newh_out):
    a = acc[0] + acc[1]
    d = jnp.sum(deg[...], axis=1)[:, None]
    newh_out[...] = jnp.tanh(sh[...] + a / jnp.maximum(d, 1.0))


def _row_spec(br):
    return pl.BlockSpec((br, D), lambda i: (i, 0))


def _acc_spec(br, width):
    return pl.BlockSpec((N_CORES, br, width), lambda i: (0, i, 0))


def _deg_spec(br):
    return pl.BlockSpec((br, NW), lambda i: (i, 0))


def _full_spec():
    return pl.BlockSpec((D, D), lambda i: (0, 0))


def _bias_spec():
    return pl.BlockSpec((1, D), lambda i: (0, 0))


def kernel(node_features, hedge_features, node_idx, hedge_idx, W_node_self,
           W_hedge2node, b_node, W_hedge_self, W_node2hedge, b_hedge):
    n_nodes, d = node_features.shape
    n_hedges = hedge_features.shape[0]
    n_inc = node_idx.shape[0]
    assert d == D and n_nodes == n_hedges and n_nodes % 1000 == 0

    assert n_inc % CH == 0
    nch_total = n_inc // CH
    nch_a = int(round(nch_total * 0.755 / N_SUB))
    nch_a += nch_a % 2
    n_a = nch_a * N_SUB * CH
    rem_ch = nch_total - nch_a * N_SUB
    assert rem_ch > 0
    nch_b = -(-rem_ch // N_SUB)
    nch_b += nch_b % 2
    assert nch_a >= 2 and nch_b >= 2
    pad_b = nch_b * N_SUB * CH - (n_inc - n_a)

    n_acc = max(n_nodes, n_hedges) + 1
    acc_rows = -(-n_acc // (N_SUB * 8)) * (N_SUB * 8)
    stripe = acc_rows // N_SUB

    def _slabs(idx, fill):
        a = idx[:n_a].reshape(N_SUB, nch_a, CH)
        b = jnp.concatenate(
            [idx[n_a:], jnp.full((pad_b,), fill, jnp.int32)])
        b = b.reshape(N_SUB, nch_b, CH)
        b = jnp.pad(b, ((0, 0), (0, nch_a - nch_b), (0, 0)),
                    constant_values=fill)
        return jnp.concatenate([a, b], axis=0)

    dummy_n = jnp.int32(n_nodes)
    dummy_h = jnp.int32(n_hedges)
    src1 = _slabs(hedge_idx, 0)
    dst1 = _slabs(node_idx, dummy_n)
    src2 = _slabs(node_idx, 0)
    dst2 = _slabs(hedge_idx, dummy_h)

    zrow = jnp.zeros((stripe, D), jnp.float32)

    bn = b_node.reshape(1, D)
    bh = b_hedge.reshape(1, D)

    br = 1000
    grid = (n_nodes // br,)
    s_n, t_h = pl.pallas_call(
        _pre_body,
        grid=grid,
        in_specs=[_row_spec(br), _row_spec(br), _full_spec(), _full_spec(),
                  _bias_spec()],
        out_specs=[_row_spec(br), _row_spec(br)],
        out_shape=[jax.ShapeDtypeStruct((n_nodes, D), jnp.float32),
                   jax.ShapeDtypeStruct((n_hedges, D), jnp.float32)],
    )(node_features, hedge_features, W_node_self, W_hedge2node, bn)

    sc_acc = _make_sc_acc(n_hedges, nch_a, nch_b, acc_rows)
    sc_deg = _make_sc_deg(nch_a, acc_rows)

    deg1, deg2 = sc_deg(dst1, dst2)
    deg1 = deg1.T
    deg2 = deg2.T

    acc1 = sc_acc(t_h, src1, dst1, zrow)

    new_node, t_n, s_h = pl.pallas_call(
        _mid_body,
        grid=grid,
        in_specs=[_row_spec(br), _acc_spec(br, D), _deg_spec(br),
                  _row_spec(br), _full_spec(), _full_spec(), _bias_spec()],
        out_specs=[_row_spec(br), _row_spec(br), _row_spec(br)],
        out_shape=[jax.ShapeDtypeStruct((n_nodes, D), jnp.float32),
                   jax.ShapeDtypeStruct((n_nodes, D), jnp.float32),
                   jax.ShapeDtypeStruct((n_hedges, D), jnp.float32)],
    )(s_n, acc1, deg1, hedge_features, W_hedge_self, W_node2hedge, bh)

    acc2 = sc_acc(t_n, src2, dst2, zrow)

    new_hedge, = pl.pallas_call(
        _post_body,
        grid=(n_hedges // br,),
        in_specs=[_row_spec(br), _acc_spec(br, D), _deg_spec(br)],
        out_specs=[_row_spec(br)],
        out_shape=[jax.ShapeDtypeStruct((n_hedges, D), jnp.float32)],
    )(s_h, acc2, deg2)

    return (new_node, new_hedge)


# speedup vs baseline: 6.6028x; 1.0874x over previous
import functools

import jax
import jax.numpy as jnp
from jax import lax
from jax.experimental import pallas as pl
from jax.experimental.pallas import tpu as pltpu
from jax.experimental.pallas import tpu_sc as plsc

N_CORES = 2
N_SUB = 16
NW = N_CORES * N_SUB
CH = 128
D = 128


def _sc_acc_body(nch_a, nch_b, acc_rows, table_hbm, src_hbm, dst_hbm,
                 zrow_hbm, acc_out, idx_d, is0, is1, gb0, gb1, acc_sh,
                 sg0, sg1, si0, si1):
    c = lax.axis_index("c")
    s = lax.axis_index("s")
    w = c * N_SUB + s
    stripe = acc_rows // N_SUB
    nch = jnp.where(c == 0, nch_a, nch_b)
    pltpu.sync_copy(zrow_hbm, acc_sh.at[pl.ds(s * stripe, stripe)])
    pltpu.sync_copy(dst_hbm.at[w], idx_d)
    plsc.subcore_barrier()

    gbufs = (gb0, gb1)
    isbufs = (is0, is1)
    sgs = (sg0, sg1)
    sis = (si0, si1)
    pltpu.sync_copy(src_hbm.at[w, 0], is0)
    pltpu.async_copy(table_hbm.at[is0], gb0, sg0)
    pltpu.async_copy(src_hbm.at[w, 1], is1, si1)

    def outer(t, carry):
        for b in range(2):
            j = t * 2 + b
            nb = 1 - b

            @pl.when(j + 1 < nch)
            def _start_next():
                pltpu.make_async_copy(src_hbm.at[w, j + 1],
                                      isbufs[nb], sis[nb]).wait()
                pltpu.async_copy(table_hbm.at[isbufs[nb]], gbufs[nb], sgs[nb])

            pltpu.make_async_copy(table_hbm.at[isbufs[b]],
                                  gbufs[b], sgs[b]).wait()

            @pl.when(j + 2 < nch)
            def _fetch_idx():
                pltpu.async_copy(src_hbm.at[w, j + 2], isbufs[b], sis[b])

            pltpu.sync_copy(gbufs[b], acc_sh.at[idx_d.at[j]], add=True)
        return carry

    lax.fori_loop(0, nch // 2, outer, 0)
    plsc.subcore_barrier()
    pltpu.sync_copy(acc_sh.at[pl.ds(s * stripe, stripe)],
                    acc_out.at[c, pl.ds(s * stripe, stripe)])


def _make_sc_acc(n_table, nch_a, nch_b, acc_rows):
    mesh = plsc.VectorSubcoreMesh(core_axis_name="c", subcore_axis_name="s")
    stripe = acc_rows // N_SUB
    return pl.kernel(
        functools.partial(_sc_acc_body, nch_a, nch_b, acc_rows),
        out_type=jax.ShapeDtypeStruct((N_CORES, acc_rows, D), jnp.float32),
        mesh=mesh,
        scratch_types=[
            pltpu.VMEM((nch_a, CH), jnp.int32),
            pltpu.VMEM((CH,), jnp.int32),
            pltpu.VMEM((CH,), jnp.int32),
            pltpu.VMEM((CH, D), jnp.float32),
            pltpu.VMEM((CH, D), jnp.float32),
            pltpu.VMEM_SHARED((acc_rows, D), jnp.float32),
            pltpu.SemaphoreType.DMA,
            pltpu.SemaphoreType.DMA,
            pltpu.SemaphoreType.DMA,
            pltpu.SemaphoreType.DMA,
        ],
        name=f"sc_acc_{n_table}",
    )


def _sc_deg_body(nch, acc_rows, dstn_hbm, dsth_hbm, degn_out, degh_out,
                 idx_n, idx_h, histn, histh):
    c = lax.axis_index("c")
    s = lax.axis_index("s")
    w = c * N_SUB + s
    zero = jnp.zeros((16,), jnp.float32)

    def initz(j, carry):
        histn[pl.ds(j * 16, 16)] = zero
        histh[pl.ds(j * 16, 16)] = zero
        return carry

    lax.fori_loop(0, acc_rows // 16, initz, 0)
    pltpu.sync_copy(dstn_hbm.at[w], idx_n)
    pltpu.sync_copy(dsth_hbm.at[w], idx_h)
    lanes = lax.iota(jnp.int32, 16)
    one = jnp.float32(1)
    zf = jnp.float32(0)

    def step(i, carry):
        j = i // (CH // 16)
        k = i % (CH // 16)
        vn = idx_n[j, pl.ds(k * 16, 16)]
        vh = idx_h[j, pl.ds(k * 16, 16)]
        for lane in range(16):
            v = vn[lane]
            plsc.addupdate(histn.at[pl.ds((v >> 4) << 4, 16)],
                           jnp.where(lanes == (v & 15), one, zf))
            u = vh[lane]
            plsc.addupdate(histh.at[pl.ds((u >> 4) << 4, 16)],
                           jnp.where(lanes == (u & 15), one, zf))
        return carry

    lax.fori_loop(0, nch * (CH // 16), step, 0)
    pltpu.sync_copy(histn, degn_out.at[w])
    pltpu.sync_copy(histh, degh_out.at[w])


def _make_sc_deg(nch, acc_rows):
    mesh = plsc.VectorSubcoreMesh(core_axis_name="c", subcore_axis_name="s")
    return pl.kernel(
        functools.partial(_sc_deg_body, nch, acc_rows),
        out_type=(
            jax.ShapeDtypeStruct((NW, acc_rows), jnp.float32),
            jax.ShapeDtypeStruct((NW, acc_rows), jnp.float32),
        ),
        mesh=mesh,
        scratch_types=[
            pltpu.VMEM((nch, CH), jnp.int32),
            pltpu.VMEM((nch, CH), jnp.int32),
            pltpu.VMEM((acc_rows,), jnp.float32),
            pltpu.VMEM((acc_rows,), jnp.float32),
        ],
        name="sc_deg",
    )


def _pre_body(nf, hf, w0, w1, bn, sn_out, th_out):
    sn_out[...] = jnp.dot(nf[...], w0[...],
                          preferred_element_type=jnp.float32) + bn[...]
    th_out[...] = jnp.dot(hf[...], w1[...], preferred_element_type=jnp.float32)


def _mid_body(sn, acc, deg, hf, wh, wn2h, bh, newn_out, tn_out, sh_out):
    a = acc[0] + acc[1]
    d = jnp.sum(deg[...], axis=1)[:, None]
    newn = jnp.tanh(sn[...] + a / jnp.maximum(d, 1.0))
    newn_out[...] = newn
    tn_out[...] = jnp.dot(newn, wn2h[...], preferred_element_type=jnp.float32)
    sh_out[...] = jnp.dot(hf[...], wh[...],
                          preferred_element_type=jnp.float32) + bh[...]


def _post_body(sh, acc, deg, newh_out):
    a = acc[0] + acc[1]
    d = jnp.sum(deg[...], axis=1)[:, None]
    newh_out[...] = jnp.tanh(sh[...] + a / jnp.maximum(d, 1.0))


def _row_spec(br):
    return pl.BlockSpec((br, D), lambda i: (i, 0))


def _acc_spec(br, width):
    return pl.BlockSpec((N_CORES, br, width), lambda i: (0, i, 0))


def _deg_spec(br):
    return pl.BlockSpec((br, NW), lambda i: (i, 0))


def _full_spec():
    return pl.BlockSpec((D, D), lambda i: (0, 0))


def _bias_spec():
    return pl.BlockSpec((1, D), lambda i: (0, 0))


def kernel(node_features, hedge_features, node_idx, hedge_idx, W_node_self,
           W_hedge2node, b_node, W_hedge_self, W_node2hedge, b_hedge):
    n_nodes, d = node_features.shape
    n_hedges = hedge_features.shape[0]
    n_inc = node_idx.shape[0]
    assert d == D and n_nodes == n_hedges and n_nodes % 1000 == 0

    assert n_inc % CH == 0
    nch_total = n_inc // CH
    nch_a = int(round(nch_total * 0.82 / N_SUB))
    nch_a += nch_a % 2
    n_a = nch_a * N_SUB * CH
    rem_ch = nch_total - nch_a * N_SUB
    assert rem_ch > 0
    nch_b = -(-rem_ch // N_SUB)
    nch_b += nch_b % 2
    assert nch_a >= 2 and nch_b >= 2
    pad_b = nch_b * N_SUB * CH - (n_inc - n_a)

    n_acc = max(n_nodes, n_hedges) + 1
    acc_rows = -(-n_acc // (N_SUB * 8)) * (N_SUB * 8)
    stripe = acc_rows // N_SUB

    def _slabs(idx, fill):
        a = idx[:n_a].reshape(N_SUB, nch_a, CH)
        b = jnp.concatenate(
            [idx[n_a:], jnp.full((pad_b,), fill, jnp.int32)])
        b = b.reshape(N_SUB, nch_b, CH)
        b = jnp.pad(b, ((0, 0), (0, nch_a - nch_b), (0, 0)),
                    constant_values=fill)
        return jnp.concatenate([a, b], axis=0)

    dummy_n = jnp.int32(n_nodes)
    dummy_h = jnp.int32(n_hedges)
    src1 = _slabs(hedge_idx, 0)
    dst1 = _slabs(node_idx, dummy_n)
    src2 = _slabs(node_idx, 0)
    dst2 = _slabs(hedge_idx, dummy_h)

    zrow = jnp.zeros((stripe, D), jnp.float32)

    bn = b_node.reshape(1, D)
    bh = b_hedge.reshape(1, D)

    br = 1000
    grid = (n_nodes // br,)
    s_n, t_h = pl.pallas_call(
        _pre_body,
        grid=grid,
        in_specs=[_row_spec(br), _row_spec(br), _full_spec(), _full_spec(),
                  _bias_spec()],
        out_specs=[_row_spec(br), _row_spec(br)],
        out_shape=[jax.ShapeDtypeStruct((n_nodes, D), jnp.float32),
                   jax.ShapeDtypeStruct((n_hedges, D), jnp.float32)],
    )(node_features, hedge_features, W_node_self, W_hedge2node, bn)

    sc_acc = _make_sc_acc(n_hedges, nch_a, nch_b, acc_rows)
    sc_deg = _make_sc_deg(nch_a, acc_rows)

    deg1, deg2 = sc_deg(dst1, dst2)
    deg1 = deg1.T
    deg2 = deg2.T

    acc1 = sc_acc(t_h, src1, dst1, zrow)

    new_node, t_n, s_h = pl.pallas_call(
        _mid_body,
        grid=grid,
        in_specs=[_row_spec(br), _acc_spec(br, D), _deg_spec(br),
                  _row_spec(br), _full_spec(), _full_spec(), _bias_spec()],
        out_specs=[_row_spec(br), _row_spec(br), _row_spec(br)],
        out_shape=[jax.ShapeDtypeStruct((n_nodes, D), jnp.float32),
                   jax.ShapeDtypeStruct((n_nodes, D), jnp.float32),
                   jax.ShapeDtypeStruct((n_hedges, D), jnp.float32)],
    )(s_n, acc1, deg1, hedge_features, W_hedge_self, W_node2hedge, bh)

    acc2 = sc_acc(t_n, src2, dst2, zrow)

    new_hedge, = pl.pallas_call(
        _post_body,
        grid=(n_hedges // br,),
        in_specs=[_row_spec(br), _acc_spec(br, D), _deg_spec(br)],
        out_specs=[_row_spec(br)],
        out_shape=[jax.ShapeDtypeStruct((n_hedges, D), jnp.float32)],
    )(s_h, acc2, deg2)

    return (new_node, new_hedge)


# speedup vs baseline: 6.8446x; 1.0366x over previous
import functools

import jax
import jax.numpy as jnp
from jax import lax
from jax.experimental import pallas as pl
from jax.experimental.pallas import tpu as pltpu
from jax.experimental.pallas import tpu_sc as plsc

N_CORES = 2
N_SUB = 16
NW = N_CORES * N_SUB
CH = 128
D = 128


def _sc_acc_body(nch_a, nch_b, acc_rows, table_hbm, src_hbm, dst_hbm,
                 zrow_hbm, acc_out, idx_d, is0, is1, gb0, gb1, acc_sh,
                 sg0, sg1, si0, si1):
    c = lax.axis_index("c")
    s = lax.axis_index("s")
    w = c * N_SUB + s
    stripe = acc_rows // N_SUB
    nch = jnp.where(c == 0, nch_a, nch_b)
    pltpu.sync_copy(zrow_hbm, acc_sh.at[pl.ds(s * stripe, stripe)])
    pltpu.sync_copy(dst_hbm.at[w], idx_d)
    plsc.subcore_barrier()

    gbufs = (gb0, gb1)
    isbufs = (is0, is1)
    sgs = (sg0, sg1)
    sis = (si0, si1)
    pltpu.sync_copy(src_hbm.at[w, 0], is0)
    pltpu.async_copy(table_hbm.at[is0], gb0, sg0)
    pltpu.async_copy(src_hbm.at[w, 1], is1, si1)

    def outer(t, carry):
        for b in range(2):
            j = t * 2 + b
            nb = 1 - b

            @pl.when(j + 1 < nch)
            def _start_next():
                pltpu.make_async_copy(src_hbm.at[w, j + 1],
                                      isbufs[nb], sis[nb]).wait()
                pltpu.async_copy(table_hbm.at[isbufs[nb]], gbufs[nb], sgs[nb])

            pltpu.make_async_copy(table_hbm.at[isbufs[b]],
                                  gbufs[b], sgs[b]).wait()

            @pl.when(j + 2 < nch)
            def _fetch_idx():
                pltpu.async_copy(src_hbm.at[w, j + 2], isbufs[b], sis[b])

            pltpu.sync_copy(gbufs[b], acc_sh.at[idx_d.at[j]], add=True)
        return carry

    lax.fori_loop(0, nch // 2, outer, 0)
    plsc.subcore_barrier()
    pltpu.sync_copy(acc_sh.at[pl.ds(s * stripe, stripe)],
                    acc_out.at[c, pl.ds(s * stripe, stripe)])


def _make_sc_acc(n_table, nch_a, nch_b, acc_rows):
    mesh = plsc.VectorSubcoreMesh(core_axis_name="c", subcore_axis_name="s")
    stripe = acc_rows // N_SUB
    return pl.kernel(
        functools.partial(_sc_acc_body, nch_a, nch_b, acc_rows),
        out_type=jax.ShapeDtypeStruct((N_CORES, acc_rows, D), jnp.float32),
        mesh=mesh,
        scratch_types=[
            pltpu.VMEM((nch_a, CH), jnp.int32),
            pltpu.VMEM((CH,), jnp.int32),
            pltpu.VMEM((CH,), jnp.int32),
            pltpu.VMEM((CH, D), jnp.float32),
            pltpu.VMEM((CH, D), jnp.float32),
            pltpu.VMEM_SHARED((acc_rows, D), jnp.float32),
            pltpu.SemaphoreType.DMA,
            pltpu.SemaphoreType.DMA,
            pltpu.SemaphoreType.DMA,
            pltpu.SemaphoreType.DMA,
        ],
        name=f"sc_acc_{n_table}",
    )


def _sc_deg_body(nch, acc_rows, dstn_hbm, dsth_hbm, degn_out, degh_out,
                 idx_n, idx_h, histn, histh):
    c = lax.axis_index("c")
    s = lax.axis_index("s")
    w = c * N_SUB + s
    zero = jnp.zeros((16,), jnp.float32)

    def initz(j, carry):
        histn[pl.ds(j * 16, 16)] = zero
        histh[pl.ds(j * 16, 16)] = zero
        return carry

    lax.fori_loop(0, acc_rows // 16, initz, 0)
    pltpu.sync_copy(dstn_hbm.at[w], idx_n)
    pltpu.sync_copy(dsth_hbm.at[w], idx_h)
    lanes = lax.iota(jnp.int32, 16)
    one = jnp.float32(1)
    zf = jnp.float32(0)

    def step(i, carry):
        j = i // (CH // 16)
        k = i % (CH // 16)
        vn = idx_n[j, pl.ds(k * 16, 16)]
        vh = idx_h[j, pl.ds(k * 16, 16)]
        for lane in range(16):
            v = vn[lane]
            plsc.addupdate(histn.at[pl.ds((v >> 4) << 4, 16)],
                           jnp.where(lanes == (v & 15), one, zf))
            u = vh[lane]
            plsc.addupdate(histh.at[pl.ds((u >> 4) << 4, 16)],
                           jnp.where(lanes == (u & 15), one, zf))
        return carry

    lax.fori_loop(0, nch * (CH // 16), step, 0)
    pltpu.sync_copy(histn, degn_out.at[w])
    pltpu.sync_copy(histh, degh_out.at[w])


def _make_sc_deg(nch, acc_rows):
    mesh = plsc.VectorSubcoreMesh(core_axis_name="c", subcore_axis_name="s")
    return pl.kernel(
        functools.partial(_sc_deg_body, nch, acc_rows),
        out_type=(
            jax.ShapeDtypeStruct((NW, acc_rows), jnp.float32),
            jax.ShapeDtypeStruct((NW, acc_rows), jnp.float32),
        ),
        mesh=mesh,
        scratch_types=[
            pltpu.VMEM((nch, CH), jnp.int32),
            pltpu.VMEM((nch, CH), jnp.int32),
            pltpu.VMEM((acc_rows,), jnp.float32),
            pltpu.VMEM((acc_rows,), jnp.float32),
        ],
        name="sc_deg",
    )


def _pre_body(nf, hf, w0, w1, bn, sn_out, th_out):
    sn_out[...] = jnp.dot(nf[...], w0[...],
                          preferred_element_type=jnp.float32) + bn[...]
    th_out[...] = jnp.dot(hf[...], w1[...], preferred_element_type=jnp.float32)


def _mid_body(sn, acc, deg, hf, wh, wn2h, bh, newn_out, tn_out, sh_out):
    a = acc[0] + acc[1]
    d = jnp.sum(deg[...], axis=1)[:, None]
    newn = jnp.tanh(sn[...] + a / jnp.maximum(d, 1.0))
    newn_out[...] = newn
    tn_out[...] = jnp.dot(newn, wn2h[...], preferred_element_type=jnp.float32)
    sh_out[...] = jnp.dot(hf[...], wh[...],
                          preferred_element_type=jnp.float32) + bh[...]


def _post_body(sh, acc, deg, newh_out):
    a = acc[0] + acc[1]
    d = jnp.sum(deg[...], axis=1)[:, None]
    newh_out[...] = jnp.tanh(sh[...] + a / jnp.maximum(d, 1.0))


def _row_spec(br):
    return pl.BlockSpec((br, D), lambda i: (i, 0))


def _acc_spec(br, width):
    return pl.BlockSpec((N_CORES, br, width), lambda i: (0, i, 0))


def _deg_spec(br):
    return pl.BlockSpec((br, NW), lambda i: (i, 0))


def _full_spec():
    return pl.BlockSpec((D, D), lambda i: (0, 0))


def _bias_spec():
    return pl.BlockSpec((1, D), lambda i: (0, 0))


def kernel(node_features, hedge_features, node_idx, hedge_idx, W_node_self,
           W_hedge2node, b_node, W_hedge_self, W_node2hedge, b_hedge):
    n_nodes, d = node_features.shape
    n_hedges = hedge_features.shape[0]
    n_inc = node_idx.shape[0]
    assert d == D and n_nodes == n_hedges and n_nodes % 1000 == 0

    assert n_inc % CH == 0
    nch_total = n_inc // CH
    nch_a = int(round(nch_total * 0.82 / N_SUB))
    nch_a += nch_a % 2
    n_a = nch_a * N_SUB * CH
    rem_ch = nch_total - nch_a * N_SUB
    assert rem_ch > 0
    nch_b = -(-rem_ch // N_SUB)
    nch_b += nch_b % 2
    assert nch_a >= 2 and nch_b >= 2
    pad_b = nch_b * N_SUB * CH - (n_inc - n_a)

    n_acc = max(n_nodes, n_hedges) + 1
    acc_rows = -(-n_acc // (N_SUB * 8)) * (N_SUB * 8)
    stripe = acc_rows // N_SUB

    def _slabs(idx, fill):
        a = idx[:n_a].reshape(N_SUB, nch_a, CH)
        b = jnp.concatenate(
            [idx[n_a:], jnp.full((pad_b,), fill, jnp.int32)])
        b = b.reshape(N_SUB, nch_b, CH)
        b = jnp.pad(b, ((0, 0), (0, nch_a - nch_b), (0, 0)),
                    constant_values=fill)
        return jnp.concatenate([a, b], axis=0)

    dummy_n = jnp.int32(n_nodes)
    dummy_h = jnp.int32(n_hedges)
    src1 = _slabs(hedge_idx, 0)
    dst1 = _slabs(node_idx, dummy_n)
    src2 = _slabs(node_idx, 0)
    dst2 = _slabs(hedge_idx, dummy_h)

    nch_e = -(-nch_total // NW)
    pad_e = nch_e * NW * CH - n_inc
    dst_e1 = jnp.concatenate(
        [node_idx, jnp.full((pad_e,), dummy_n, jnp.int32)]).reshape(
            NW, nch_e, CH)
    dst_e2 = jnp.concatenate(
        [hedge_idx, jnp.full((pad_e,), dummy_h, jnp.int32)]).reshape(
            NW, nch_e, CH)

    zrow = jnp.zeros((stripe, D), jnp.float32)

    bn = b_node.reshape(1, D)
    bh = b_hedge.reshape(1, D)

    br = 1000
    grid = (n_nodes // br,)
    s_n, t_h = pl.pallas_call(
        _pre_body,
        grid=grid,
        in_specs=[_row_spec(br), _row_spec(br), _full_spec(), _full_spec(),
                  _bias_spec()],
        out_specs=[_row_spec(br), _row_spec(br)],
        out_shape=[jax.ShapeDtypeStruct((n_nodes, D), jnp.float32),
                   jax.ShapeDtypeStruct((n_hedges, D), jnp.float32)],
    )(node_features, hedge_features, W_node_self, W_hedge2node, bn)

    sc_acc = _make_sc_acc(n_hedges, nch_a, nch_b, acc_rows)
    sc_deg = _make_sc_deg(nch_e, acc_rows)

    deg1, deg2 = sc_deg(dst_e1, dst_e2)
    deg1 = deg1.T
    deg2 = deg2.T

    acc1 = sc_acc(t_h, src1, dst1, zrow)

    new_node, t_n, s_h = pl.pallas_call(
        _mid_body,
        grid=grid,
        in_specs=[_row_spec(br), _acc_spec(br, D), _deg_spec(br),
                  _row_spec(br), _full_spec(), _full_spec(), _bias_spec()],
        out_specs=[_row_spec(br), _row_spec(br), _row_spec(br)],
        out_shape=[jax.ShapeDtypeStruct((n_nodes, D), jnp.float32),
                   jax.ShapeDtypeStruct((n_nodes, D), jnp.float32),
                   jax.ShapeDtypeStruct((n_hedges, D), jnp.float32)],
    )(s_n, acc1, deg1, hedge_features, W_hedge_self, W_node2hedge, bh)

    acc2 = sc_acc(t_n, src2, dst2, zrow)

    new_hedge, = pl.pallas_call(
        _post_body,
        grid=(n_hedges // br,),
        in_specs=[_row_spec(br), _acc_spec(br, D), _deg_spec(br)],
        out_specs=[_row_spec(br)],
        out_shape=[jax.ShapeDtypeStruct((n_hedges, D), jnp.float32)],
    )(s_h, acc2, deg2)

    return (new_node, new_hedge)
